# baseline (speedup 1.0000x reference)
import functools

import jax
import jax.numpy as jnp
from jax import lax
from jax.experimental import pallas as pl
from jax.experimental.pallas import tpu as pltpu
from jax.experimental.pallas import tpu_sc as plsc

N = 500
NP = 512
F = 16
H = 32
B = 16
T = 50
BT = B * T
E = 8000
EP = 8192
PER = EP // 16
SL = NP * NP // 16
STRIP = 80
NSTRIP = BT // STRIP


def _densify_body(src_h, dst_h, w_h, z_h, out_h, src_v, dst_v, w_v, idx_v, acc):
    cid = lax.axis_index("c")
    sid = lax.axis_index("s")

    zoff = pl.multiple_of(sid * SL, 8)
    pltpu.sync_copy(z_h, acc.at[pl.ds(zoff, SL)])
    plsc.subcore_barrier()

    eoff = pl.multiple_of(sid * PER, 8)
    pltpu.sync_copy(src_h.at[cid, pl.ds(eoff, PER)], src_v)
    pltpu.sync_copy(dst_h.at[cid, pl.ds(eoff, PER)], dst_v)
    pltpu.sync_copy(w_h.at[cid, pl.ds(eoff, PER)], w_v)

    for j in range(PER // 128):
        for u in range(8):
            t = j * 8 + u
            s16 = src_v[pl.ds(t * 16, 16)]
            d16 = dst_v[pl.ds(t * 16, 16)]
            idx_v[j, pl.ds(u * 16, 16)] = s16 * NP + d16

    for j in range(PER // 128):
        pltpu.sync_copy(w_v.at[pl.ds(j * 128, 128)],
                        acc.at[idx_v.at[j]], add=True)
    plsc.subcore_barrier()

    pltpu.sync_copy(acc.at[pl.ds(zoff, SL)], out_h.at[cid, pl.ds(zoff, SL)])


def _densify(src_all, dst_all, w_all, zeros_sl):
    mesh = plsc.VectorSubcoreMesh(core_axis_name="c", subcore_axis_name="s")
    k = functools.partial(
        pl.kernel,
        mesh=mesh,
        out_type=jax.ShapeDtypeStruct((2, NP * NP), jnp.float32),
        scratch_types=[
            pltpu.VMEM((PER,), jnp.int32),
            pltpu.VMEM((PER,), jnp.int32),
            pltpu.VMEM((PER,), jnp.float32),
            pltpu.VMEM((PER // 128, 128), jnp.int32),
            pltpu.VMEM_SHARED((NP * NP,), jnp.float32),
        ],
    )(_densify_body)
    return k(src_all, dst_all, w_all, zeros_sl)


CH = 64
NCH = NP // CH


def _prep_body(atsp_ref, atfn_ref, wch_ref, wg_ref, bch_ref, bg_ref,
               w4_ref, bih_ref, dsp_ref, dgn_ref, c_ref, const_ref):
    d_id = pl.program_id(0)
    c_id = pl.program_id(1)
    j = pl.program_id(2)

    @pl.when((d_id == 0) & (c_id == 0) & (j == 0))
    def _():
        deg = jnp.sum(atsp_ref[...], axis=0, keepdims=True)
        dsp_ref[...] = jnp.where(
            deg > 0, lax.rsqrt(jnp.where(deg > 0, deg, 1.0)), 0.0)
        degg = jnp.sum(atfn_ref[...], axis=0, keepdims=True) + 1.0
        dgn_ref[...] = lax.rsqrt(degg)

    z = jnp.zeros((F, H), jnp.float32)
    mcs = [
        jnp.concatenate([wch_ref[0], z], axis=1),
        jnp.concatenate([wch_ref[1], z], axis=1),
        jnp.concatenate([wch_ref[2], z], axis=1),
        jnp.concatenate([z, wg_ref[...]], axis=1),
    ]
    mc = sum(mcs[i] * (c_id == i).astype(jnp.float32) for i in range(4))

    w4 = w4_ref[0]
    c_ref[...] = lax.dot_general(
        mc, w4, (((1,), (0,)), ((), ())),
        preferred_element_type=jnp.float32)[None]

    @pl.when(c_id == 0)
    def _():
        sw = jnp.sum(w4, axis=1)
        bias2 = jnp.concatenate(
            [bch_ref[...], bg_ref[...]])[None, :]
        part = lax.dot_general(bias2, sw, (((1,), (0,)), ((), ())),
                               preferred_element_type=jnp.float32)

        @pl.when(j == 0)
        def _():
            const_ref[...] = (part + bih_ref[0])[None]

        @pl.when(j > 0)
        def _():
            const_ref[...] = const_ref[...] + part[None]


def _prep(at_sp, at_fn, w_cheb, w_gcn, b_cheb, b_gcn, w4_both, bih_both):
    return pl.pallas_call(
        _prep_body,
        grid=(2, 4, NCH),
        in_specs=[
            pl.BlockSpec((NP, NP), lambda d, c, j: (0, 0)),
            pl.BlockSpec((NP, NP), lambda d, c, j: (0, 0)),
            pl.BlockSpec((3, F, H), lambda d, c, j: (0, 0, 0)),
            pl.BlockSpec((F, H), lambda d, c, j: (0, 0)),
            pl.BlockSpec((H,), lambda d, c, j: (0,)),
            pl.BlockSpec((H,), lambda d, c, j: (0,)),
            pl.BlockSpec((1, 64, CH, 96), lambda d, c, j: (d, 0, j, 0)),
            pl.BlockSpec((1, 1, 96), lambda d, c, j: (d, 0, 0)),
        ],
        out_specs=[
            pl.BlockSpec((1, NP), lambda d, c, j: (0, 0)),
            pl.BlockSpec((1, NP), lambda d, c, j: (0, 0)),
            pl.BlockSpec((1, F, CH, 96), lambda d, c, j: (d, 0, c * NCH + j, 0)),
            pl.BlockSpec((1, 1, 96), lambda d, c, j: (d, 0, 0)),
        ],
        out_shape=[
            jax.ShapeDtypeStruct((1, NP), jnp.float32),
            jax.ShapeDtypeStruct((1, NP), jnp.float32),
            jax.ShapeDtypeStruct((2, F, 4 * NP, 96), jnp.float32),
            jax.ShapeDtypeStruct((2, 1, 96), jnp.float32),
        ],
    )(at_sp, at_fn, w_cheb, w_gcn, b_cheb, b_gcn, w4_both, bih_both)


def _heavy_body(x_ref, atsp_ref, atfn_ref, dsp_ref, dgn_ref,
                cf_ref, cb_ref, constf_ref, constb_ref, gif_ref, gib_ref):
    x = x_ref[...].reshape(F * STRIP, NP)
    atsp = atsp_ref[...]
    atfn = atfn_ref[...]
    d = dsp_ref[...]
    dg = dgn_ref[...]

    def prop_sp(v):
        return -lax.dot_general(
            v * d, atsp, (((1,), (0,)), ((), ())),
            preferred_element_type=jnp.float32) * d

    tx1 = prop_sp(x)
    tx2 = 2.0 * prop_sp(tx1) - x
    g = lax.dot_general(
        x * dg, atfn, (((1,), (0,)), ((), ())),
        preferred_element_type=jnp.float32) * dg + x * (dg * dg)

    u2 = jnp.concatenate([x, tx1, tx2, g], axis=1)

    for (c_ref, const_ref, out_ref) in ((cf_ref, constf_ref, gif_ref),
                                        (cb_ref, constb_ref, gib_ref)):
        acc = jnp.broadcast_to(const_ref[...], (STRIP, 96))
        for f in range(F):
            acc = acc + lax.dot_general(
                u2[f * STRIP:(f + 1) * STRIP], c_ref[f],
                (((1,), (0,)), ((), ())),
                preferred_element_type=jnp.float32)
        out_ref[...] = acc


def _heavy(x0, at_sp, at_fn, dsp, dgn, c2f, c2b, constf, constb):
    return pl.pallas_call(
        _heavy_body,
        grid=(NSTRIP,),
        in_specs=[
            pl.BlockSpec((F, STRIP, NP), lambda i: (0, i, 0)),
            pl.BlockSpec((NP, NP), lambda i: (0, 0)),
            pl.BlockSpec((NP, NP), lambda i: (0, 0)),
            pl.BlockSpec((1, NP), lambda i: (0, 0)),
            pl.BlockSpec((1, NP), lambda i: (0, 0)),
            pl.BlockSpec((F, 4 * NP, 96), lambda i: (0, 0, 0)),
            pl.BlockSpec((F, 4 * NP, 96), lambda i: (0, 0, 0)),
            pl.BlockSpec((1, 96), lambda i: (0, 0)),
            pl.BlockSpec((1, 96), lambda i: (0, 0)),
        ],
        out_specs=[
            pl.BlockSpec((STRIP, 96), lambda i: (i, 0)),
            pl.BlockSpec((STRIP, 96), lambda i: (i, 0)),
        ],
        out_shape=[
            jax.ShapeDtypeStruct((BT, 96), jnp.float32),
            jax.ShapeDtypeStruct((BT, 96), jnp.float32),
        ],
    )(x0, at_sp, at_fn, dsp, dgn, c2f, c2b, constf, constb)


def _gru_body(gif_ref, gib_ref, whf_ref, bhf_ref, whb_ref, bhb_ref,
              wat_ref, bat_ref, wcl_ref, bcl_ref, out_ref, go_ref):
    whf = whf_ref[...]
    bhf = bhf_ref[...]
    whb = whb_ref[...]
    bhb = bhb_ref[...]

    def gru_step(gi, h, wh, bh):
        gh = jnp.dot(h, wh, preferred_element_type=jnp.float32) + bh
        r = jax.nn.sigmoid(gi[:, 0:H] + gh[:, 0:H])
        z = jax.nn.sigmoid(gi[:, H:2 * H] + gh[:, H:2 * H])
        n = jnp.tanh(gi[:, 2 * H:] + r * gh[:, 2 * H:])
        return (1.0 - z) * n + z * h

    def step(t, carry):
        hf, hb = carry
        gf = gif_ref[pl.ds(t, 1)][0]
        hf2 = gru_step(gf, hf, whf, bhf)
        tb = T - 1 - t
        gb = gib_ref[pl.ds(tb, 1)][0]
        hb2 = gru_step(gb, hb, whb, bhb)
        go_ref[pl.ds(t, 1), :, 0:H] = hf2[None]
        go_ref[pl.ds(tb, 1), :, H:2 * H] = hb2[None]
        return (hf2, hb2)

    z0 = jnp.zeros((B, H), jnp.float32)
    lax.fori_loop(0, T, step, (z0, z0))

    go = go_ref[...]
    s = jnp.tanh(
        lax.dot_general(go, wat_ref[...], (((2,), (0,)), ((), ())),
                        preferred_element_type=jnp.float32)
        + bat_ref[...])[:, :, 0]
    mx = jnp.max(s, axis=0, keepdims=True)
    ex = jnp.exp(s - mx)
    a = ex / jnp.sum(ex, axis=0, keepdims=True)
    ctx = jnp.sum(a[:, :, None] * go, axis=0)
    out_ref[...] = jax.nn.sigmoid(
        jnp.dot(ctx, wcl_ref[...], preferred_element_type=jnp.float32)
        + bcl_ref[...])


def _gru(gif, gib, whfT, bhhf, whbT, bhhb, w_attn, b_attn, w_cls, b_cls):
    return pl.pallas_call(
        _gru_body,
        out_shape=jax.ShapeDtypeStruct((B, 1), jnp.float32),
        scratch_shapes=[pltpu.VMEM((T, B, 2 * H), jnp.float32)],
    )(gif, gib, whfT, bhhf, whbT, bhhb, w_attn, b_attn, w_cls, b_cls)


def kernel(x, spatial_edge_index, spatial_edge_weight,
           functional_edge_index, functional_edge_weight,
           W_cheb, b_cheb, W_gcn, b_gcn,
           Wih_f, Whh_f, bih_f, bhh_f, Wih_b, Whh_b, bih_b, bhh_b,
           W_attn, b_attn, W_cls, b_cls):
    f32 = jnp.float32
    padE = EP - E
    src_all = jnp.stack([
        jnp.pad(spatial_edge_index[0], (0, padE)),
        jnp.pad(functional_edge_index[0], (0, padE)),
    ]).astype(jnp.int32)
    dst_all = jnp.stack([
        jnp.pad(spatial_edge_index[1], (0, padE)),
        jnp.pad(functional_edge_index[1], (0, padE)),
    ]).astype(jnp.int32)
    w_all = jnp.stack([
        jnp.pad(spatial_edge_weight.astype(f32), (0, padE)),
        jnp.pad(functional_edge_weight.astype(f32), (0, padE)),
    ])
    at_all = _densify(src_all, dst_all, w_all, jnp.zeros((SL,), f32))
    at_sp = at_all[0].reshape(NP, NP)
    at_fn = at_all[1].reshape(NP, NP)

    x0 = jnp.pad(
        jnp.transpose(x, (3, 0, 2, 1)).reshape(F, BT, N),
        ((0, 0), (0, 0), (0, NP - N))).astype(f32)

    def mk_w4(wih):
        w4 = wih.T.reshape(N, 2 * H, 96)
        w4 = jnp.pad(w4, ((0, NP - N), (0, 0), (0, 0)))
        return jnp.transpose(w4, (1, 0, 2))

    w4_both = jnp.stack([mk_w4(Wih_f), mk_w4(Wih_b)]).astype(f32)
    bih_both = jnp.stack([bih_f, bih_b]).astype(f32)[:, None, :]

    dsp, dgn, c_all, const_all = _prep(
        at_sp, at_fn, W_cheb.astype(f32), W_gcn.astype(f32),
        b_cheb.astype(f32), b_gcn.astype(f32), w4_both, bih_both)

    gi_f, gi_b = _heavy(x0, at_sp, at_fn, dsp, dgn,
                        c_all[0], c_all[1], const_all[0], const_all[1])

    gif_t = gi_f.reshape(B, T, 96).transpose(1, 0, 2)
    gib_t = gi_b.reshape(B, T, 96).transpose(1, 0, 2)

    return _gru(gif_t, gib_t,
                Whh_f.T.astype(f32), bhh_f.astype(f32),
                Whh_b.T.astype(f32), bhh_b.astype(f32),
                W_attn.astype(f32), b_attn.astype(f32),
                W_cls.astype(f32), b_cls.astype(f32))


# speedup vs baseline: 1.0631x; 1.0631x over previous
import functools

import jax
import jax.numpy as jnp
from jax import lax
from jax.experimental import pallas as pl
from jax.experimental.pallas import tpu as pltpu
from jax.experimental.pallas import tpu_sc as plsc

N = 500
NP = 512
F = 16
H = 32
B = 16
T = 50
BT = B * T
E = 8000
EP = 8192
PER = EP // 16
SL = NP * NP // 16
STRIP = 80
NSTRIP = BT // STRIP


def _densify_body(src_h, dst_h, w_h, z_h, out_h, src_v, dst_v, w_v, idx_v, acc):
    cid = lax.axis_index("c")
    sid = lax.axis_index("s")

    zoff = pl.multiple_of(sid * SL, 8)
    pltpu.sync_copy(z_h, acc.at[pl.ds(zoff, SL)])
    plsc.subcore_barrier()

    eoff = pl.multiple_of(sid * PER, 8)
    pltpu.sync_copy(src_h.at[cid, pl.ds(eoff, PER)], src_v)
    pltpu.sync_copy(dst_h.at[cid, pl.ds(eoff, PER)], dst_v)
    pltpu.sync_copy(w_h.at[cid, pl.ds(eoff, PER)], w_v)

    for j in range(PER // 128):
        for u in range(8):
            t = j * 8 + u
            s16 = src_v[pl.ds(t * 16, 16)]
            d16 = dst_v[pl.ds(t * 16, 16)]
            idx_v[j, pl.ds(u * 16, 16)] = s16 * NP + d16

    for j in range(PER // 128):
        pltpu.sync_copy(w_v.at[pl.ds(j * 128, 128)],
                        acc.at[idx_v.at[j]], add=True)
    plsc.subcore_barrier()

    pltpu.sync_copy(acc.at[pl.ds(zoff, SL)], out_h.at[cid, pl.ds(zoff, SL)])


def _densify(src_all, dst_all, w_all, zeros_sl):
    mesh = plsc.VectorSubcoreMesh(core_axis_name="c", subcore_axis_name="s")
    k = functools.partial(
        pl.kernel,
        mesh=mesh,
        out_type=jax.ShapeDtypeStruct((2, NP * NP), jnp.float32),
        scratch_types=[
            pltpu.VMEM((PER,), jnp.int32),
            pltpu.VMEM((PER,), jnp.int32),
            pltpu.VMEM((PER,), jnp.float32),
            pltpu.VMEM((PER // 128, 128), jnp.int32),
            pltpu.VMEM_SHARED((NP * NP,), jnp.float32),
        ],
    )(_densify_body)
    return k(src_all, dst_all, w_all, zeros_sl)


CHF = 256
NJ = NP // CHF


def _fold_body(m_ref, w4_ref, bias2_ref, bih_ref, c_ref, const_ref):
    c_id = pl.program_id(1)
    j = pl.program_id(2)

    w4 = w4_ref[0]
    c_ref[...] = lax.dot_general(
        m_ref[...], w4, (((1,), (0,)), ((), ())),
        preferred_element_type=jnp.float32)[None]

    @pl.when(c_id == 0)
    def _():
        sw = jnp.sum(w4, axis=1)
        part = lax.dot_general(bias2_ref[...], sw, (((1,), (0,)), ((), ())),
                               preferred_element_type=jnp.float32)

        @pl.when(j == 0)
        def _():
            const_ref[...] = (part + bih_ref[0])[None]

        @pl.when(j > 0)
        def _():
            const_ref[...] = const_ref[...] + part[None]


def _fold(m, w4_both, bias2, bih_both):
    return pl.pallas_call(
        _fold_body,
        grid=(2, 4, NJ),
        in_specs=[
            pl.BlockSpec((F, 64), lambda d, c, j: (c, 0)),
            pl.BlockSpec((1, 64, CHF, 96), lambda d, c, j: (d, 0, j, 0)),
            pl.BlockSpec((1, 64), lambda d, c, j: (0, 0)),
            pl.BlockSpec((1, 1, 96), lambda d, c, j: (d, 0, 0)),
        ],
        out_specs=[
            pl.BlockSpec((1, F, CHF, 96), lambda d, c, j: (d, 0, c * NJ + j, 0)),
            pl.BlockSpec((1, 1, 96), lambda d, c, j: (d, 0, 0)),
        ],
        out_shape=[
            jax.ShapeDtypeStruct((2, F, 4 * NP, 96), jnp.float32),
            jax.ShapeDtypeStruct((2, 1, 96), jnp.float32),
        ],
    )(m, w4_both, bias2, bih_both)


def _deg_body(atsp_ref, atfn_ref, dsp_ref, dgn_ref):
    deg = jnp.sum(atsp_ref[...], axis=0, keepdims=True)
    dsp_ref[...] = jnp.where(
        deg > 0, lax.rsqrt(jnp.where(deg > 0, deg, 1.0)), 0.0)
    degg = jnp.sum(atfn_ref[...], axis=0, keepdims=True) + 1.0
    dgn_ref[...] = lax.rsqrt(degg)


def _deg(at_sp, at_fn):
    return pl.pallas_call(
        _deg_body,
        out_shape=[
            jax.ShapeDtypeStruct((1, NP), jnp.float32),
            jax.ShapeDtypeStruct((1, NP), jnp.float32),
        ],
    )(at_sp, at_fn)


def _heavy_body(x_ref, atsp_ref, atfn_ref, dsp_ref, dgn_ref,
                cf_ref, cb_ref, constf_ref, constb_ref, gif_ref, gib_ref):
    x = x_ref[...].reshape(F * STRIP, NP)
    atsp = atsp_ref[...]
    atfn = atfn_ref[...]
    d = dsp_ref[...]
    dg = dgn_ref[...]

    def prop_sp(v):
        return -lax.dot_general(
            v * d, atsp, (((1,), (0,)), ((), ())),
            preferred_element_type=jnp.float32) * d

    tx1 = prop_sp(x)
    tx2 = 2.0 * prop_sp(tx1) - x
    g = lax.dot_general(
        x * dg, atfn, (((1,), (0,)), ((), ())),
        preferred_element_type=jnp.float32) * dg + x * (dg * dg)

    u2 = jnp.concatenate([x, tx1, tx2, g], axis=1)

    for (c_ref, const_ref, out_ref) in ((cf_ref, constf_ref, gif_ref),
                                        (cb_ref, constb_ref, gib_ref)):
        acc = jnp.broadcast_to(const_ref[...], (STRIP, 96))
        for f in range(F):
            acc = acc + lax.dot_general(
                u2[f * STRIP:(f + 1) * STRIP], c_ref[f],
                (((1,), (0,)), ((), ())),
                preferred_element_type=jnp.float32)
        out_ref[...] = acc


def _heavy(x0, at_sp, at_fn, dsp, dgn, c2f, c2b, constf, constb):
    return pl.pallas_call(
        _heavy_body,
        grid=(NSTRIP,),
        in_specs=[
            pl.BlockSpec((F, STRIP, NP), lambda i: (0, i, 0)),
            pl.BlockSpec((NP, NP), lambda i: (0, 0)),
            pl.BlockSpec((NP, NP), lambda i: (0, 0)),
            pl.BlockSpec((1, NP), lambda i: (0, 0)),
            pl.BlockSpec((1, NP), lambda i: (0, 0)),
            pl.BlockSpec((F, 4 * NP, 96), lambda i: (0, 0, 0)),
            pl.BlockSpec((F, 4 * NP, 96), lambda i: (0, 0, 0)),
            pl.BlockSpec((1, 96), lambda i: (0, 0)),
            pl.BlockSpec((1, 96), lambda i: (0, 0)),
        ],
        out_specs=[
            pl.BlockSpec((STRIP, 96), lambda i: (i, 0)),
            pl.BlockSpec((STRIP, 96), lambda i: (i, 0)),
        ],
        out_shape=[
            jax.ShapeDtypeStruct((BT, 96), jnp.float32),
            jax.ShapeDtypeStruct((BT, 96), jnp.float32),
        ],
    )(x0, at_sp, at_fn, dsp, dgn, c2f, c2b, constf, constb)


def _gru_body(gif_ref, gib_ref, whf_ref, bhf_ref, whb_ref, bhb_ref,
              wat_ref, bat_ref, wcl_ref, bcl_ref, out_ref, go_ref):
    whf = whf_ref[...]
    bhf = bhf_ref[...]
    whb = whb_ref[...]
    bhb = bhb_ref[...]

    def gru_step(gi, h, wh, bh):
        gh = jnp.dot(h, wh, preferred_element_type=jnp.float32) + bh
        r = jax.nn.sigmoid(gi[:, 0:H] + gh[:, 0:H])
        z = jax.nn.sigmoid(gi[:, H:2 * H] + gh[:, H:2 * H])
        n = jnp.tanh(gi[:, 2 * H:] + r * gh[:, 2 * H:])
        return (1.0 - z) * n + z * h

    def step(t, carry):
        hf, hb = carry
        gf = gif_ref[pl.ds(t, 1)][0]
        hf2 = gru_step(gf, hf, whf, bhf)
        tb = T - 1 - t
        gb = gib_ref[pl.ds(tb, 1)][0]
        hb2 = gru_step(gb, hb, whb, bhb)
        go_ref[pl.ds(t, 1), :, 0:H] = hf2[None]
        go_ref[pl.ds(tb, 1), :, H:2 * H] = hb2[None]
        return (hf2, hb2)

    z0 = jnp.zeros((B, H), jnp.float32)
    lax.fori_loop(0, T, step, (z0, z0))

    go = go_ref[...]
    s = jnp.tanh(
        lax.dot_general(go, wat_ref[...], (((2,), (0,)), ((), ())),
                        preferred_element_type=jnp.float32)
        + bat_ref[...])[:, :, 0]
    mx = jnp.max(s, axis=0, keepdims=True)
    ex = jnp.exp(s - mx)
    a = ex / jnp.sum(ex, axis=0, keepdims=True)
    ctx = jnp.sum(a[:, :, None] * go, axis=0)
    out_ref[...] = jax.nn.sigmoid(
        jnp.dot(ctx, wcl_ref[...], preferred_element_type=jnp.float32)
        + bcl_ref[...])


def _gru(gif, gib, whfT, bhhf, whbT, bhhb, w_attn, b_attn, w_cls, b_cls):
    return pl.pallas_call(
        _gru_body,
        out_shape=jax.ShapeDtypeStruct((B, 1), jnp.float32),
        scratch_shapes=[pltpu.VMEM((T, B, 2 * H), jnp.float32)],
    )(gif, gib, whfT, bhhf, whbT, bhhb, w_attn, b_attn, w_cls, b_cls)


def kernel(x, spatial_edge_index, spatial_edge_weight,
           functional_edge_index, functional_edge_weight,
           W_cheb, b_cheb, W_gcn, b_gcn,
           Wih_f, Whh_f, bih_f, bhh_f, Wih_b, Whh_b, bih_b, bhh_b,
           W_attn, b_attn, W_cls, b_cls):
    f32 = jnp.float32
    padE = EP - E
    src_all = jnp.stack([
        jnp.pad(spatial_edge_index[0], (0, padE)),
        jnp.pad(functional_edge_index[0], (0, padE)),
    ]).astype(jnp.int32)
    dst_all = jnp.stack([
        jnp.pad(spatial_edge_index[1], (0, padE)),
        jnp.pad(functional_edge_index[1], (0, padE)),
    ]).astype(jnp.int32)
    w_all = jnp.stack([
        jnp.pad(spatial_edge_weight.astype(f32), (0, padE)),
        jnp.pad(functional_edge_weight.astype(f32), (0, padE)),
    ])
    at_all = _densify(src_all, dst_all, w_all, jnp.zeros((SL,), f32))
    at_sp = at_all[0].reshape(NP, NP)
    at_fn = at_all[1].reshape(NP, NP)

    x0 = jnp.pad(
        jnp.transpose(x, (3, 0, 2, 1)).reshape(F, BT, N),
        ((0, 0), (0, 0), (0, NP - N))).astype(f32)

    def mk_w4(wih):
        w4 = wih.T.reshape(N, 2 * H, 96)
        w4 = jnp.pad(w4, ((0, NP - N), (0, 0), (0, 0)))
        return jnp.transpose(w4, (1, 0, 2))

    w4_both = jnp.stack([mk_w4(Wih_f), mk_w4(Wih_b)]).astype(f32)
    bih_both = jnp.stack([bih_f, bih_b]).astype(f32)[:, None, :]

    zfh = jnp.zeros((F, H), f32)
    wch = W_cheb.astype(f32)
    m = jnp.concatenate([
        jnp.concatenate([wch[0], zfh], axis=1),
        jnp.concatenate([wch[1], zfh], axis=1),
        jnp.concatenate([wch[2], zfh], axis=1),
        jnp.concatenate([zfh, W_gcn.astype(f32)], axis=1),
    ], axis=0)
    bias2 = jnp.concatenate(
        [b_cheb.astype(f32), b_gcn.astype(f32)])[None, :]

    c_all, const_all = _fold(m, w4_both, bias2, bih_both)
    dsp, dgn = _deg(at_sp, at_fn)

    gi_f, gi_b = _heavy(x0, at_sp, at_fn, dsp, dgn,
                        c_all[0], c_all[1], const_all[0], const_all[1])

    gif_t = gi_f.reshape(B, T, 96).transpose(1, 0, 2)
    gib_t = gi_b.reshape(B, T, 96).transpose(1, 0, 2)

    return _gru(gif_t, gib_t,
                Whh_f.T.astype(f32), bhh_f.astype(f32),
                Whh_b.T.astype(f32), bhh_b.astype(f32),
                W_attn.astype(f32), b_attn.astype(f32),
                W_cls.astype(f32), b_cls.astype(f32))


# speedup vs baseline: 1.3850x; 1.3028x over previous
import functools

import jax
import jax.numpy as jnp
from jax import lax
from jax.experimental import pallas as pl
from jax.experimental.pallas import tpu as pltpu
from jax.experimental.pallas import tpu_sc as plsc

N = 500
NP = 512
F = 16
H = 32
B = 16
T = 50
BT = B * T
E = 8000
EP = 8192
PER = EP // 16
SL = NP * NP // 16
STRIP = 80
NSTRIP = BT // STRIP


def _densify_body(src_h, dst_h, w_h, z_h, out_h,
                  src_v, dst_v, w_v, idx_v, acc):
    cid = lax.axis_index("c")
    sid = lax.axis_index("s")

    zoff = pl.multiple_of(sid * SL, 8)
    pltpu.sync_copy(z_h, acc.at[pl.ds(zoff, SL)])
    plsc.subcore_barrier()

    eoff = pl.multiple_of(sid * PER, 8)
    pltpu.sync_copy(src_h.at[cid, pl.ds(eoff, PER)], src_v)
    pltpu.sync_copy(dst_h.at[cid, pl.ds(eoff, PER)], dst_v)
    pltpu.sync_copy(w_h.at[cid, pl.ds(eoff, PER)], w_v)

    for j in range(PER // 128):
        for u in range(8):
            t = j * 8 + u
            s16 = src_v[pl.ds(t * 16, 16)]
            d16 = dst_v[pl.ds(t * 16, 16)]
            idx_v[j, pl.ds(u * 16, 16)] = s16 * NP + d16

    for j in range(PER // 128):
        pltpu.sync_copy(w_v.at[pl.ds(j * 128, 128)],
                        acc.at[idx_v.at[j]], add=True)
    plsc.subcore_barrier()

    pltpu.sync_copy(acc.at[pl.ds(zoff, SL)], out_h.at[cid, pl.ds(zoff, SL)])


def _densify(src_all, dst_all, w_all, zeros_sl):
    mesh = plsc.VectorSubcoreMesh(core_axis_name="c", subcore_axis_name="s")
    k = functools.partial(
        pl.kernel,
        mesh=mesh,
        out_type=jax.ShapeDtypeStruct((2, NP * NP), jnp.float32),
        scratch_types=[
            pltpu.VMEM((PER,), jnp.int32),
            pltpu.VMEM((PER,), jnp.int32),
            pltpu.VMEM((PER,), jnp.float32),
            pltpu.VMEM((PER // 128, 128), jnp.int32),
            pltpu.VMEM_SHARED((NP * NP,), jnp.float32),
        ],
    )(_densify_body)
    return k(src_all, dst_all, w_all, zeros_sl)


def _fold_body(m_ref, w5_ref, bias2_ref, bih_ref, c_ref, const_ref):
    c_id = pl.program_id(0)

    w5 = w5_ref[...]
    c_ref[...] = lax.dot_general(
        m_ref[...], w5, (((1,), (2,)), ((), ())),
        preferred_element_type=jnp.float32)[None]

    @pl.when(c_id == 0)
    def _():
        sw = jnp.sum(w5, axis=1)
        part = lax.dot_general(bias2_ref[...], sw, (((1,), (1,)), ((), ())),
                               preferred_element_type=jnp.float32)
        const_ref[...] = part + bih_ref[...]


def _fold(m, w5, bias2, bih):
    return pl.pallas_call(
        _fold_body,
        grid=(4,),
        in_specs=[
            pl.BlockSpec((F, 64), lambda c: (c, 0)),
            pl.BlockSpec((96, N, 64), lambda c: (0, 0, 0)),
            pl.BlockSpec((1, 64), lambda c: (0, 0)),
            pl.BlockSpec((1, 96), lambda c: (0, 0)),
        ],
        out_specs=[
            pl.BlockSpec((1, F, 96, N), lambda c: (c, 0, 0, 0)),
            pl.BlockSpec((1, 96), lambda c: (0, 0)),
        ],
        out_shape=[
            jax.ShapeDtypeStruct((4, F, 96, N), jnp.float32),
            jax.ShapeDtypeStruct((1, 96), jnp.float32),
        ],
    )(m, w5, bias2, bih)


def _deg_body(at_ref, dsp_ref, dgn_ref):
    deg = jnp.sum(at_ref[0], axis=0, keepdims=True)
    dsp_ref[...] = jnp.where(
        deg > 0, lax.rsqrt(jnp.where(deg > 0, deg, 1.0)), 0.0)
    degg = jnp.sum(at_ref[1], axis=0, keepdims=True) + 1.0
    dgn_ref[...] = lax.rsqrt(degg)


def _deg(at3):
    return pl.pallas_call(
        _deg_body,
        out_shape=[
            jax.ShapeDtypeStruct((1, NP), jnp.float32),
            jax.ShapeDtypeStruct((1, NP), jnp.float32),
        ],
    )(at3)


def _heavy_body(x_ref, at_ref, dsp_ref, dgn_ref,
                cf_ref, cb_ref, constf_ref, constb_ref, gif_ref, gib_ref):
    x = x_ref[...].reshape(F * STRIP, NP)
    atsp = at_ref[0]
    atfn = at_ref[1]
    d = dsp_ref[...]
    dg = dgn_ref[...]

    def prop_sp(v):
        return -lax.dot_general(
            v * d, atsp, (((1,), (0,)), ((), ())),
            preferred_element_type=jnp.float32) * d

    tx1 = prop_sp(x)
    tx2 = 2.0 * prop_sp(tx1) - x
    g = lax.dot_general(
        x * dg, atfn, (((1,), (0,)), ((), ())),
        preferred_element_type=jnp.float32) * dg + x * (dg * dg)

    terms = [v.reshape(F, STRIP, NP)[:, :, :N] for v in (x, tx1, tx2, g)]

    for (c_ref, const_ref, out_ref) in ((cf_ref, constf_ref, gif_ref),
                                        (cb_ref, constb_ref, gib_ref)):
        acc = jnp.broadcast_to(const_ref[...], (STRIP, 96))
        for c in range(4):
            acc = acc + jnp.sum(lax.dot_general(
                terms[c], c_ref[c],
                (((2,), (2,)), ((0,), (0,))),
                preferred_element_type=jnp.float32), axis=0)
        out_ref[...] = acc


def _heavy(x0, at3, dsp, dgn, c2f, c2b, constf, constb):
    return pl.pallas_call(
        _heavy_body,
        grid=(NSTRIP,),
        in_specs=[
            pl.BlockSpec((F, STRIP, NP), lambda i: (0, i, 0)),
            pl.BlockSpec((2, NP, NP), lambda i: (0, 0, 0)),
            pl.BlockSpec((1, NP), lambda i: (0, 0)),
            pl.BlockSpec((1, NP), lambda i: (0, 0)),
            pl.BlockSpec((4, F, 96, N), lambda i: (0, 0, 0, 0)),
            pl.BlockSpec((4, F, 96, N), lambda i: (0, 0, 0, 0)),
            pl.BlockSpec((1, 96), lambda i: (0, 0)),
            pl.BlockSpec((1, 96), lambda i: (0, 0)),
        ],
        out_specs=[
            pl.BlockSpec((STRIP, 96), lambda i: (i, 0)),
            pl.BlockSpec((STRIP, 96), lambda i: (i, 0)),
        ],
        out_shape=[
            jax.ShapeDtypeStruct((BT, 96), jnp.float32),
            jax.ShapeDtypeStruct((BT, 96), jnp.float32),
        ],
    )(x0, at3, dsp, dgn, c2f, c2b, constf, constb)


def _gru_body(gif_ref, gib_ref, wblk_ref, bblk_ref,
              wat_ref, bat_ref, wcl_ref, bcl_ref, out_ref, gc_ref, go_ref):
    gtf = jnp.transpose(gif_ref[...], (1, 0, 2))
    gtb = jnp.transpose(gib_ref[...], (1, 0, 2))
    gc_ref[...] = jnp.concatenate([gtf, gtb], axis=1)

    wblk = wblk_ref[...]
    bblk = bblk_ref[...]
    rowf = (lax.broadcasted_iota(jnp.int32, (2 * B, 1), 0) < B)

    def step(t, hc):
        ga = gc_ref[pl.ds(t, 1)][0]
        gb = gc_ref[pl.ds(T - 1 - t, 1)][0]
        gi = jnp.where(rowf, ga, gb)
        gh = jnp.dot(hc, wblk, preferred_element_type=jnp.float32) + bblk
        h = jnp.where(rowf, hc[:, 0:H], hc[:, H:2 * H])
        r = jax.nn.sigmoid(gi[:, 0:H] + gh[:, 0:H])
        z = jax.nn.sigmoid(gi[:, H:2 * H] + gh[:, H:2 * H])
        n = jnp.tanh(gi[:, 2 * H:] + r * gh[:, 2 * H:])
        h2 = (1.0 - z) * n + z * h
        go_ref[pl.ds(t, 1), :, 0:H] = h2[None, 0:B]
        go_ref[pl.ds(T - 1 - t, 1), :, H:2 * H] = h2[None, B:]
        zero = jnp.zeros_like(h2)
        return jnp.concatenate([jnp.where(rowf, h2, zero),
                                jnp.where(rowf, zero, h2)], axis=1)

    lax.fori_loop(0, T, step, jnp.zeros((2 * B, 2 * H), jnp.float32))

    go = go_ref[...]
    s = jnp.tanh(
        lax.dot_general(go, wat_ref[...], (((2,), (0,)), ((), ())),
                        preferred_element_type=jnp.float32)
        + bat_ref[...])[:, :, 0]
    mx = jnp.max(s, axis=0, keepdims=True)
    ex = jnp.exp(s - mx)
    a = ex / jnp.sum(ex, axis=0, keepdims=True)
    ctx = jnp.sum(a[:, :, None] * go, axis=0)
    out_ref[...] = jax.nn.sigmoid(
        jnp.dot(ctx, wcl_ref[...], preferred_element_type=jnp.float32)
        + bcl_ref[...])


def _gru(gif, gib, wblk, bblk, w_attn, b_attn, w_cls, b_cls):
    return pl.pallas_call(
        _gru_body,
        out_shape=jax.ShapeDtypeStruct((B, 1), jnp.float32),
        scratch_shapes=[pltpu.VMEM((T, 2 * B, 96), jnp.float32),
                        pltpu.VMEM((T, B, 2 * H), jnp.float32)],
    )(gif, gib, wblk, bblk, w_attn, b_attn, w_cls, b_cls)


def kernel(x, spatial_edge_index, spatial_edge_weight,
           functional_edge_index, functional_edge_weight,
           W_cheb, b_cheb, W_gcn, b_gcn,
           Wih_f, Whh_f, bih_f, bhh_f, Wih_b, Whh_b, bih_b, bhh_b,
           W_attn, b_attn, W_cls, b_cls):
    f32 = jnp.float32
    padE = EP - E
    src_all = jnp.stack([
        jnp.pad(spatial_edge_index[0], (0, padE)),
        jnp.pad(functional_edge_index[0], (0, padE)),
    ]).astype(jnp.int32)
    dst_all = jnp.stack([
        jnp.pad(spatial_edge_index[1], (0, padE)),
        jnp.pad(functional_edge_index[1], (0, padE)),
    ]).astype(jnp.int32)
    w_all = jnp.stack([
        jnp.pad(spatial_edge_weight.astype(f32), (0, padE)),
        jnp.pad(functional_edge_weight.astype(f32), (0, padE)),
    ])
    at_all = _densify(src_all, dst_all, w_all, jnp.zeros((SL,), f32))
    at3 = at_all.reshape(2, NP, NP)

    x0 = jnp.pad(
        jnp.transpose(x, (3, 0, 2, 1)).reshape(F, BT, N),
        ((0, 0), (0, 0), (0, NP - N))).astype(f32)

    zfh = jnp.zeros((F, H), f32)
    wch = W_cheb.astype(f32)
    m = jnp.concatenate([
        jnp.concatenate([wch[0], zfh], axis=1),
        jnp.concatenate([wch[1], zfh], axis=1),
        jnp.concatenate([wch[2], zfh], axis=1),
        jnp.concatenate([zfh, W_gcn.astype(f32)], axis=1),
    ], axis=0)
    bias2 = jnp.concatenate(
        [b_cheb.astype(f32), b_gcn.astype(f32)])[None, :]

    cf, constf = _fold(m, Wih_f.astype(f32).reshape(96, N, 64),
                       bias2, bih_f.astype(f32)[None, :])
    cb, constb = _fold(m, Wih_b.astype(f32).reshape(96, N, 64),
                       bias2, bih_b.astype(f32)[None, :])
    dsp, dgn = _deg(at3)

    gi_f, gi_b = _heavy(x0, at3, dsp, dgn, cf, cb, constf, constb)

    wblk = jnp.concatenate(
        [Whh_f.T.astype(f32), Whh_b.T.astype(f32)], axis=0)
    bblk = jnp.concatenate(
        [jnp.broadcast_to(bhh_f.astype(f32), (B, 96)),
         jnp.broadcast_to(bhh_b.astype(f32), (B, 96))], axis=0)

    return _gru(gi_f.reshape(B, T, 96), gi_b.reshape(B, T, 96),
                wblk, bblk,
                W_attn.astype(f32), b_attn.astype(f32),
                W_cls.astype(f32), b_cls.astype(f32))
